# baseline (speedup 1.0000x reference)
import functools

import jax
import jax.numpy as jnp
from jax import lax
from jax.experimental import pallas as pl
from jax.experimental.pallas import tpu as pltpu
from jax.experimental.pallas import tpu_sc as plsc

N_NODES = 10000
N_EDGES = 320000
D_FEAT = 128

_LANES = 16
_NC = 2
_NS = 16
_NW = _NC * _NS
_CHUNK = 80
_EPW = N_EDGES // _NW
_NCHUNKS = _EPW // _CHUNK
_WB = 624
_TAIL = N_NODES - _NS * _WB
_ZR = 16

_GATHER_DNUMS = lax.GatherDimensionNumbers(
    offset_dims=(), collapsed_slice_dims=(0,), start_index_map=(0,))


def _splat(vec, lane):
    idx = jnp.full((_LANES, 1), lane, dtype=jnp.int32)
    return lax.gather(vec, idx, _GATHER_DNUMS, (1,),
                      mode=lax.GatherScatterMode.PROMISE_IN_BOUNDS)


@functools.partial(
    pl.kernel,
    out_type=jax.ShapeDtypeStruct((_NC, N_NODES, D_FEAT), jnp.float32),
    mesh=plsc.VectorSubcoreMesh(core_axis_name="c", subcore_axis_name="s"),
    scratch_types=[
        pltpu.VMEM((_CHUNK,), jnp.int32),
        pltpu.VMEM((_CHUNK,), jnp.int32),
        pltpu.VMEM((_CHUNK,), jnp.int32),
        pltpu.VMEM((_CHUNK,), jnp.int32),
        pltpu.VMEM((_CHUNK,), jnp.int32),
        pltpu.VMEM((_CHUNK,), jnp.int32),
        pltpu.VMEM((_CHUNK,), jnp.int32),
        pltpu.VMEM((_CHUNK,), jnp.int32),
        pltpu.VMEM((_CHUNK,), jnp.float32),
        pltpu.VMEM((_CHUNK,), jnp.float32),
        pltpu.VMEM((_CHUNK,), jnp.float32),
        pltpu.VMEM((_CHUNK,), jnp.float32),
        pltpu.VMEM((_CHUNK, D_FEAT), jnp.float32),
        pltpu.VMEM((_CHUNK, D_FEAT), jnp.float32),
        pltpu.VMEM((_CHUNK, D_FEAT), jnp.float32),
        pltpu.VMEM((_CHUNK, D_FEAT), jnp.float32),
        pltpu.VMEM((_ZR, D_FEAT), jnp.float32),
        pltpu.VMEM_SHARED((N_NODES, D_FEAT), jnp.float32),
        pltpu.SemaphoreType.DMA,
        pltpu.SemaphoreType.DMA,
        pltpu.SemaphoreType.DMA,
        pltpu.SemaphoreType.DMA,
        pltpu.SemaphoreType.DMA,
        pltpu.SemaphoreType.DMA,
        pltpu.SemaphoreType.DMA,
        pltpu.SemaphoreType.DMA,
    ],
)
def _sc_spmm(row_hbm, col_hbm, val_hbm, emb_hbm, out_hbm,
             col0, col1, col2, col3, row0, row1, row2, row3,
             val0, val1, val2, val3, rows0, rows1, rows2, rows3,
             zero_v, acc_sh, semi0, semi1, semr0, semr1,
             semg0, semg1, sems0, sems1):
    c = lax.axis_index("c")
    s = lax.axis_index("s")
    wid = s * _NC + c
    ebase = wid * _EPW

    col = (col0, col1, col2, col3)
    row = (row0, row1, row2, row3)
    val = (val0, val1, val2, val3)
    rows = (rows0, rows1, rows2, rows3)
    semi = (semi0, semi1)
    semr = (semr0, semr1)
    semg = (semg0, semg1)
    sems = (sems0, sems1)
    dummy = pl.ds(0, _CHUNK)

    def colval_fetch(j, b, a):
        base = ebase + j * _CHUNK
        pltpu.async_copy(col_hbm.at[pl.ds(base, _CHUNK)], col[b], semi[a])
        pltpu.async_copy(val_hbm.at[pl.ds(base, _CHUNK)], val[b], semi[a])

    def colval_wait(b, a):
        pltpu.make_async_copy(col_hbm.at[dummy], col[b], semi[a]).wait()
        pltpu.make_async_copy(val_hbm.at[dummy], val[b], semi[a]).wait()

    def row_fetch(j, b, a):
        base = ebase + j * _CHUNK
        pltpu.async_copy(row_hbm.at[pl.ds(base, _CHUNK)], row[b], semr[a])

    def row_wait(b, a):
        pltpu.make_async_copy(row_hbm.at[dummy], row[b], semr[a]).wait()

    _H = _CHUNK // 2

    def gather_start(b, a):
        pltpu.async_copy(emb_hbm.at[col[b].at[pl.ds(0, _H)]],
                         rows[b].at[pl.ds(0, _H)], semg[a])
        pltpu.async_copy(emb_hbm.at[col[b].at[pl.ds(_H, _H)]],
                         rows[b].at[pl.ds(_H, _H)], semg[a])

    def gather_wait(b, a):
        pltpu.make_async_copy(emb_hbm.at[pl.ds(0, _H)],
                              rows[b].at[pl.ds(0, _H)], semg[a]).wait()
        pltpu.make_async_copy(emb_hbm.at[pl.ds(0, _H)],
                              rows[b].at[pl.ds(_H, _H)], semg[a]).wait()

    def scatter_start(b, a):
        pltpu.async_copy(rows[b], acc_sh.at[row[b]], sems[a], add=True)

    def scatter_wait(b, a):
        pltpu.make_async_copy(rows[b], acc_sh.at[dummy], sems[a]).wait()

    def scale(b):
        valb, rowsb = val[b], rows[b]

        def gbody(g, carry):
            vals = valb[pl.ds(g * _LANES, _LANES)]
            for l in range(_LANES):
                sv = _splat(vals, l)
                r = rowsb.at[g * _LANES + l]
                for j in range(D_FEAT // _LANES):
                    sl = pl.ds(j * _LANES, _LANES)
                    r[sl] = r[sl] * sv
            return carry

        lax.fori_loop(0, _CHUNK // _LANES, gbody, None)

    zf = jnp.zeros((_LANES,), jnp.float32)
    for r in range(_ZR):
        for j in range(D_FEAT // _LANES):
            zero_v.at[r][pl.ds(j * _LANES, _LANES)] = zf
    z0 = pl.multiple_of(s * _WB, 8)
    for b in range(_WB // _ZR):
        pltpu.sync_copy(zero_v, acc_sh.at[pl.ds(z0 + b * _ZR, _ZR)])

    @pl.when(s == 0)
    def _zero_tail():
        pltpu.sync_copy(zero_v, acc_sh.at[pl.ds(_NS * _WB, _TAIL)])

    plsc.subcore_barrier()

    colval_fetch(0, 0, 0)
    row_fetch(0, 0, 0)
    colval_wait(0, 0)
    gather_start(0, 0)
    colval_fetch(1, 1, 1)
    row_fetch(1, 1, 1)

    def chunk_step(t, j, pos, guard_scatter_wait, guard_fetch):
        b = pos % 4
        a = pos % 2
        b1 = (pos + 1) % 4
        a1 = (pos + 1) % 2
        b2 = (pos + 2) % 4
        colval_wait(b1, a1)
        gather_start(b1, a1)
        gather_wait(b, a)
        scale(b)
        if guard_scatter_wait:
            @pl.when(t >= 1)
            def _w():
                scatter_wait(b, a)
        else:
            scatter_wait(b, a)
        row_wait(b, a)
        scatter_start(b, a)
        if guard_fetch:
            @pl.when(t <= (_NCHUNKS - 1) // 4 - 2)
            def _f():
                colval_fetch(j + 2, b2, a)
                row_fetch(j + 2, b2, a)
        else:
            colval_fetch(j + 2, b2, a)
            row_fetch(j + 2, b2, a)

    def quad(t, carry):
        j0 = 4 * t
        chunk_step(t, j0 + 0, 0, True, False)
        chunk_step(t, j0 + 1, 1, True, False)
        chunk_step(t, j0 + 2, 2, False, False)
        chunk_step(t, j0 + 3, 3, False, True)
        return carry

    lax.fori_loop(0, (_NCHUNKS - 1) // 4, quad, None)

    gather_wait(0, 0)
    scale(0)
    scatter_wait(0, 0)
    row_wait(0, 0)
    scatter_start(0, 0)
    scatter_wait(1, 1)
    scatter_wait(0, 0)

    plsc.subcore_barrier()
    r0 = pl.multiple_of(s * _WB, 8)
    pltpu.sync_copy(acc_sh.at[pl.ds(r0, _WB)],
                    out_hbm.at[c].at[pl.ds(r0, _WB)])

    @pl.when(s == 0)
    def _write_tail():
        pltpu.sync_copy(acc_sh.at[pl.ds(_NS * _WB, _TAIL)],
                        out_hbm.at[c].at[pl.ds(_NS * _WB, _TAIL)])


def _add_body(a_ref, b_ref, o_ref):
    o_ref[...] = a_ref[0] + b_ref[0]


def _combine(partials):
    blk = 1000
    return pl.pallas_call(
        _add_body,
        grid=(N_NODES // blk,),
        in_specs=[pl.BlockSpec((1, blk, D_FEAT), lambda i: (0, i, 0)),
                  pl.BlockSpec((1, blk, D_FEAT), lambda i: (1, i, 0))],
        out_specs=pl.BlockSpec((blk, D_FEAT), lambda i: (i, 0)),
        out_shape=jax.ShapeDtypeStruct((N_NODES, D_FEAT), jnp.float32),
    )(partials, partials)


def kernel(adj_indices, adj_values, embeds):
    adj = adj_indices.astype(jnp.int32)
    partials = _sc_spmm(adj[0], adj[1], adj_values, embeds)
    return _combine(partials)


# speedup vs baseline: 1.0283x; 1.0283x over previous
import functools

import jax
import jax.numpy as jnp
from jax import lax
from jax.experimental import pallas as pl
from jax.experimental.pallas import tpu as pltpu
from jax.experimental.pallas import tpu_sc as plsc

N_NODES = 10000
N_EDGES = 320000
D_FEAT = 128

_LANES = 16
_NC = 2
_NS = 16
_NW = _NC * _NS
_CHUNK = 80
_EPW = N_EDGES // _NW
_NCHUNKS = _EPW // _CHUNK
_WB = 624
_TAIL = N_NODES - _NS * _WB

_GATHER_DNUMS = lax.GatherDimensionNumbers(
    offset_dims=(), collapsed_slice_dims=(0,), start_index_map=(0,))


def _splat(vec, lane):
    idx = jnp.full((_LANES, 1), lane, dtype=jnp.int32)
    return lax.gather(vec, idx, _GATHER_DNUMS, (1,),
                      mode=lax.GatherScatterMode.PROMISE_IN_BOUNDS)


@functools.partial(
    pl.kernel,
    out_type=jax.ShapeDtypeStruct((_NC, N_NODES, D_FEAT), jnp.float32),
    mesh=plsc.VectorSubcoreMesh(core_axis_name="c", subcore_axis_name="s"),
    scratch_types=[
        pltpu.VMEM((_EPW,), jnp.int32),
        pltpu.VMEM((_CHUNK,), jnp.int32),
        pltpu.VMEM((_CHUNK,), jnp.int32),
        pltpu.VMEM((_EPW,), jnp.float32),
        pltpu.VMEM((_CHUNK, D_FEAT), jnp.float32),
        pltpu.VMEM((_CHUNK, D_FEAT), jnp.float32),
        pltpu.VMEM_SHARED((N_NODES, D_FEAT), jnp.float32),
        pltpu.SemaphoreType.DMA,
        pltpu.SemaphoreType.DMA,
        pltpu.SemaphoreType.DMA,
        pltpu.SemaphoreType.DMA,
        pltpu.SemaphoreType.DMA,
        pltpu.SemaphoreType.DMA,
    ],
)
def _sc_spmm(row_hbm, col_hbm, val_hbm, emb_hbm, out_hbm,
             col_v, rowb0, rowb1, val_v, rows0, rows1,
             acc_sh, semg0, semg1, sems0, sems1, semr0, semr1):
    c = lax.axis_index("c")
    s = lax.axis_index("s")
    wid = s * _NC + c
    ebase = wid * _EPW

    rows = (rows0, rows1)
    rowb = (rowb0, rowb1)
    semg = (semg0, semg1)
    sems = (sems0, sems1)
    semr = (semr0, semr1)

    def gather_start(k, b):
        idx = col_v.at[pl.ds(k * _CHUNK, _CHUNK)]
        pltpu.async_copy(emb_hbm.at[idx], rows[b], semg[b])

    def gather_wait(b):
        pltpu.make_async_copy(emb_hbm.at[pl.ds(0, _CHUNK)], rows[b],
                              semg[b]).wait()

    def row_fetch(k, b):
        base = ebase + k * _CHUNK
        pltpu.async_copy(row_hbm.at[pl.ds(base, _CHUNK)], rowb[b], semr[b])

    def row_wait(b):
        pltpu.make_async_copy(row_hbm.at[pl.ds(0, _CHUNK)], rowb[b],
                              semr[b]).wait()

    def scatter_start(k, b):
        pltpu.async_copy(rows[b], acc_sh.at[rowb[b]], sems[b], add=True)

    def scatter_wait(b):
        pltpu.make_async_copy(rows[b], acc_sh.at[pl.ds(0, _CHUNK)],
                              sems[b]).wait()

    def scale(k, b):
        rowsb = rows[b]

        def gbody(g, carry):
            vals = val_v[pl.ds(k * _CHUNK + g * _LANES, _LANES)]
            for l in range(_LANES):
                sv = _splat(vals, l)
                r = rowsb.at[g * _LANES + l]
                for j in range(D_FEAT // _LANES):
                    sl = pl.ds(j * _LANES, _LANES)
                    r[sl] = r[sl] * sv
            return carry

        lax.fori_loop(0, _CHUNK // _LANES, gbody, None)

    pltpu.sync_copy(col_hbm.at[pl.ds(ebase, _EPW)], col_v)
    pltpu.sync_copy(val_hbm.at[pl.ds(ebase, _EPW)], val_v)

    zf = jnp.zeros((_LANES,), jnp.float32)

    def zbody(r, carry):
        rr = rows0.at[r]
        for j in range(D_FEAT // _LANES):
            rr[pl.ds(j * _LANES, _LANES)] = zf
        return carry

    lax.fori_loop(0, _CHUNK, zbody, None)
    z0 = pl.multiple_of(s * _WB, 8)
    for i in range(7):
        pltpu.sync_copy(rows0, acc_sh.at[pl.ds(z0 + i * _CHUNK, _CHUNK)])
    pltpu.sync_copy(rows0.at[pl.ds(0, _WB - 7 * _CHUNK)],
                    acc_sh.at[pl.ds(z0 + 7 * _CHUNK, _WB - 7 * _CHUNK)])

    @pl.when(s == 0)
    def _zero_tail():
        pltpu.sync_copy(rows0.at[pl.ds(0, _TAIL)],
                        acc_sh.at[pl.ds(_NS * _WB, _TAIL)])

    plsc.subcore_barrier()

    row_fetch(0, 0)
    gather_start(0, 0)

    def chunk_step(t, j, p, guard_scatter_wait):
        q = 1 - p
        if guard_scatter_wait:
            @pl.when(t >= 1)
            def _w():
                scatter_wait(q)
        else:
            scatter_wait(q)
        row_fetch(j + 1, q)
        gather_start(j + 1, q)
        gather_wait(p)
        scale(j, p)
        row_wait(p)
        scatter_start(j, p)

    def pair(t, carry):
        j0 = 2 * t
        chunk_step(t, j0, 0, True)
        chunk_step(t, j0 + 1, 1, False)
        return carry

    lax.fori_loop(0, (_NCHUNKS - 1) // 2, pair, None)

    scatter_wait(1)
    gather_wait(0)
    scale(_NCHUNKS - 1, 0)
    row_wait(0)
    scatter_start(_NCHUNKS - 1, 0)
    scatter_wait(0)

    plsc.subcore_barrier()
    r0 = pl.multiple_of(s * _WB, 8)
    pltpu.sync_copy(acc_sh.at[pl.ds(r0, _WB)],
                    out_hbm.at[c].at[pl.ds(r0, _WB)])

    @pl.when(s == 0)
    def _write_tail():
        pltpu.sync_copy(acc_sh.at[pl.ds(_NS * _WB, _TAIL)],
                        out_hbm.at[c].at[pl.ds(_NS * _WB, _TAIL)])


def _add_body(a_ref, b_ref, o_ref):
    o_ref[...] = a_ref[0] + b_ref[0]


def _combine(partials):
    blk = 1000
    return pl.pallas_call(
        _add_body,
        grid=(N_NODES // blk,),
        in_specs=[pl.BlockSpec((1, blk, D_FEAT), lambda i: (0, i, 0)),
                  pl.BlockSpec((1, blk, D_FEAT), lambda i: (1, i, 0))],
        out_specs=pl.BlockSpec((blk, D_FEAT), lambda i: (i, 0)),
        out_shape=jax.ShapeDtypeStruct((N_NODES, D_FEAT), jnp.float32),
    )(partials, partials)


def kernel(adj_indices, adj_values, embeds):
    adj = adj_indices.astype(jnp.int32)
    partials = _sc_spmm(adj[0], adj[1], adj_values, embeds)
    return _combine(partials)


# speedup vs baseline: 1.1597x; 1.1278x over previous
import functools

import jax
import jax.numpy as jnp
from jax import lax
from jax.experimental import pallas as pl
from jax.experimental.pallas import tpu as pltpu
from jax.experimental.pallas import tpu_sc as plsc

N_NODES = 10000
N_EDGES = 320000
D_FEAT = 128

_LANES = 16
_NC = 2
_NS = 16
_NW = _NC * _NS
_CHUNK = 128
_EPW = N_EDGES // _NW
_NFULL = _EPW // _CHUNK
_TAILE = _EPW - _NFULL * _CHUNK
_WB = 624
_TAIL = N_NODES - _NS * _WB

_GATHER_DNUMS = lax.GatherDimensionNumbers(
    offset_dims=(), collapsed_slice_dims=(0,), start_index_map=(0,))


def _splat(vec, lane):
    idx = jnp.full((_LANES, 1), lane, dtype=jnp.int32)
    return lax.gather(vec, idx, _GATHER_DNUMS, (1,),
                      mode=lax.GatherScatterMode.PROMISE_IN_BOUNDS)


@functools.partial(
    pl.kernel,
    out_type=jax.ShapeDtypeStruct((_NC, N_NODES, D_FEAT), jnp.float32),
    mesh=plsc.VectorSubcoreMesh(core_axis_name="c", subcore_axis_name="s"),
    scratch_types=[
        pltpu.VMEM((_EPW,), jnp.int32),
        pltpu.VMEM((_CHUNK,), jnp.int32),
        pltpu.VMEM((_CHUNK,), jnp.int32),
        pltpu.VMEM((_TAILE,), jnp.int32),
        pltpu.VMEM((_CHUNK,), jnp.float32),
        pltpu.VMEM((_CHUNK,), jnp.float32),
        pltpu.VMEM((_CHUNK, D_FEAT), jnp.float32),
        pltpu.VMEM((_CHUNK, D_FEAT), jnp.float32),
        pltpu.VMEM_SHARED((N_NODES, D_FEAT), jnp.float32),
        pltpu.SemaphoreType.DMA,
        pltpu.SemaphoreType.DMA,
        pltpu.SemaphoreType.DMA,
        pltpu.SemaphoreType.DMA,
        pltpu.SemaphoreType.DMA,
        pltpu.SemaphoreType.DMA,
    ],
)
def _sc_spmm(adj_hbm, val_hbm, emb_hbm, out_hbm,
             col_v, rowb0, rowb1, rowtail, valb0, valb1, rows0, rows1,
             acc_sh, semg0, semg1, sems0, sems1, semr0, semr1):
    c = lax.axis_index("c")
    s = lax.axis_index("s")
    wid = s * _NC + c
    ebase = wid * _EPW

    rows = (rows0, rows1)
    rowb = (rowb0, rowb1)
    valb = (valb0, valb1)
    semg = (semg0, semg1)
    sems = (sems0, sems1)
    semr = (semr0, semr1)

    def gather_start(k, b, n=_CHUNK):
        idx = col_v.at[pl.ds(k * _CHUNK, n)]
        pltpu.async_copy(emb_hbm.at[idx], rows[b].at[pl.ds(0, n)], semg[b])

    def gather_wait(b, n=_CHUNK):
        pltpu.make_async_copy(emb_hbm.at[pl.ds(0, n)],
                              rows[b].at[pl.ds(0, n)], semg[b]).wait()

    def rowval_fetch(k, b):
        base = ebase + k * _CHUNK
        pltpu.async_copy(adj_hbm.at[pl.ds(base, _CHUNK)], rowb[b], semr[b])
        pltpu.async_copy(val_hbm.at[pl.ds(base, _CHUNK)], valb[b], semr[b])

    def rowval_wait(b):
        pltpu.make_async_copy(adj_hbm.at[pl.ds(0, _CHUNK)], rowb[b],
                              semr[b]).wait()
        pltpu.make_async_copy(val_hbm.at[pl.ds(0, _CHUNK)], valb[b],
                              semr[b]).wait()

    def scatter_start(b):
        pltpu.async_copy(rows[b], acc_sh.at[rowb[b]], sems[b], add=True)

    def scatter_wait(b, n=_CHUNK):
        pltpu.make_async_copy(rows[b].at[pl.ds(0, n)],
                              acc_sh.at[pl.ds(0, n)], sems[b]).wait()

    def scale(b, ngroups=_CHUNK // _LANES):
        rowsb, valbb = rows[b], valb[b]

        def gbody(g, carry):
            vals = valbb[pl.ds(g * _LANES, _LANES)]
            for l in range(_LANES):
                sv = _splat(vals, l)
                r = rowsb.at[g * _LANES + l]
                for j in range(D_FEAT // _LANES):
                    sl = pl.ds(j * _LANES, _LANES)
                    r[sl] = r[sl] * sv
            return carry

        lax.fori_loop(0, ngroups, gbody, None)

    pltpu.sync_copy(adj_hbm.at[pl.ds(N_EDGES + ebase, _EPW)], col_v)

    zf = jnp.zeros((_LANES,), jnp.float32)

    def zbody(r, carry):
        rr = rows0.at[r]
        for j in range(D_FEAT // _LANES):
            rr[pl.ds(j * _LANES, _LANES)] = zf
        return carry

    lax.fori_loop(0, _CHUNK, zbody, None)
    z0 = pl.multiple_of(s * _WB, 8)
    for i in range(4):
        pltpu.sync_copy(rows0, acc_sh.at[pl.ds(z0 + i * _CHUNK, _CHUNK)])
    pltpu.sync_copy(rows0.at[pl.ds(0, _WB - 4 * _CHUNK)],
                    acc_sh.at[pl.ds(z0 + 4 * _CHUNK, _WB - 4 * _CHUNK)])

    @pl.when(s == 0)
    def _zero_tail():
        pltpu.sync_copy(rows0.at[pl.ds(0, _TAIL)],
                        acc_sh.at[pl.ds(_NS * _WB, _TAIL)])

    plsc.subcore_barrier()

    rowval_fetch(0, 0)
    gather_start(0, 0)

    def chunk_step(t, j, p, guard_scatter_wait, guard_fetch):
        q = 1 - p
        if guard_scatter_wait:
            @pl.when(t >= 1)
            def _w():
                scatter_wait(q)
        else:
            scatter_wait(q)
        if guard_fetch:
            @pl.when(t <= _NFULL // 2 - 2)
            def _f():
                rowval_fetch(j + 1, q)
                gather_start(j + 1, q)
        else:
            rowval_fetch(j + 1, q)
            gather_start(j + 1, q)
        gather_wait(p)
        scale(p)
        rowval_wait(p)
        scatter_start(p)

    def pair(t, carry):
        j0 = 2 * t
        chunk_step(t, j0, 0, True, False)
        chunk_step(t, j0 + 1, 1, False, True)
        return carry

    lax.fori_loop(0, _NFULL // 2, pair, None)

    tbase = ebase + _NFULL * _CHUNK
    pltpu.async_copy(adj_hbm.at[pl.ds(tbase, _TAILE)], rowtail, semr0)
    pltpu.async_copy(val_hbm.at[pl.ds(tbase, _TAILE)],
                     valb0.at[pl.ds(0, _TAILE)], semr0)
    scatter_wait(1)
    gather_start(_NFULL, 0, _TAILE)
    gather_wait(0, _TAILE)
    scale(0, _TAILE // _LANES)
    pltpu.make_async_copy(adj_hbm.at[pl.ds(0, _TAILE)], rowtail,
                          semr0).wait()
    pltpu.make_async_copy(val_hbm.at[pl.ds(0, _TAILE)],
                          valb0.at[pl.ds(0, _TAILE)], semr0).wait()
    pltpu.async_copy(rows0.at[pl.ds(0, _TAILE)], acc_sh.at[rowtail], sems0,
                     add=True)
    scatter_wait(0, _TAILE)

    plsc.subcore_barrier()
    r0 = pl.multiple_of(s * _WB, 8)
    pltpu.sync_copy(acc_sh.at[pl.ds(r0, _WB)],
                    out_hbm.at[c].at[pl.ds(r0, _WB)])

    @pl.when(s == 0)
    def _write_tail():
        pltpu.sync_copy(acc_sh.at[pl.ds(_NS * _WB, _TAIL)],
                        out_hbm.at[c].at[pl.ds(_NS * _WB, _TAIL)])


def _add_body(a_ref, b_ref, o_ref):
    o_ref[...] = a_ref[0] + b_ref[0]


def _combine(partials):
    blk = 1000
    return pl.pallas_call(
        _add_body,
        grid=(N_NODES // blk,),
        in_specs=[pl.BlockSpec((1, blk, D_FEAT), lambda i: (0, i, 0)),
                  pl.BlockSpec((1, blk, D_FEAT), lambda i: (1, i, 0))],
        out_specs=pl.BlockSpec((blk, D_FEAT), lambda i: (i, 0)),
        out_shape=jax.ShapeDtypeStruct((N_NODES, D_FEAT), jnp.float32),
    )(partials, partials)


def kernel(adj_indices, adj_values, embeds):
    adj_flat = adj_indices.astype(jnp.int32).reshape(-1)
    partials = _sc_spmm(adj_flat, adj_values, embeds)
    return _combine(partials)
